# baseline (speedup 1.0000x reference)
import functools

import jax
import jax.numpy as jnp
from jax import lax
from jax.experimental import pallas as pl
from jax.experimental.pallas import tpu as pltpu
from jax.experimental.pallas import tpu_sc as plsc

VOCAB = 1000000
EMBED = 64
BATCH = 16384
SEQ = 50

NC = 2
NS = 16
NW = NC * NS

BPW = BATCH // NW
BLK = 128
BLKS_PER_S = BPW // BLK
NBLK = SEQ * BLKS_PER_S
L = 16


@functools.partial(
    pl.kernel,
    out_type=jax.ShapeDtypeStruct((SEQ, EMBED, BATCH), jnp.float32),
    mesh=plsc.VectorSubcoreMesh(core_axis_name="c", subcore_axis_name="s"),
    scratch_types=[
        pltpu.VMEM((SEQ, BPW), jnp.int32),
        pltpu.VMEM((2, BLK, 129), jnp.float32),
        pltpu.VMEM((2, EMBED, BLK), jnp.float32),
        pltpu.VMEM((2, BLK), jnp.int32),
        pltpu.VMEM((2, BLK), jnp.int32),
        pltpu.SemaphoreType.DMA((2,)),
        pltpu.SemaphoreType.DMA((2,)),
    ],
    compiler_params=pltpu.CompilerParams(needs_layout_passes=False),
)
def _embed_lookup(tab2, idx_t, out, idxall, rows, stage, idxhalf, colb,
                  gsem, ssem):
    wid = lax.axis_index("s") * NC + lax.axis_index("c")
    w0 = wid * BPW
    iota16 = lax.iota(jnp.int32, L)
    rowvecs = [iota16 + g * L for g in range(BLK // L)]

    def stage_idx(s, carry):
        pltpu.sync_copy(idx_t.at[s, pl.ds(w0, BPW)], idxall.at[s])
        return carry

    lax.fori_loop(0, SEQ, stage_idx, 0)

    def prep(k, p):
        s = k // BLKS_PER_S
        c0 = (k % BLKS_PER_S) * BLK
        for g in range(BLK // L):
            v = idxall[s, pl.ds(c0 + g * L, L)]
            idxhalf[p, pl.ds(g * L, L)] = v >> 1
            colb[p, pl.ds(g * L, L)] = (v & 1) * EMBED
        pltpu.async_copy(
            tab2.at[idxhalf.at[p]], rows.at[p, :, pl.ds(0, 128)],
            gsem.at[p])

    def wait_gather(p):
        pltpu.make_async_copy(
            tab2.at[idxhalf.at[p]], rows.at[p, :, pl.ds(0, 128)],
            gsem.at[p]).wait()

    def fire_store(k, p):
        s = k // BLKS_PER_S
        b0 = w0 + (k % BLKS_PER_S) * BLK
        pltpu.async_copy(
            stage.at[p], out.at[s, :, pl.ds(b0, BLK)], ssem.at[p])

    def wait_store(p):
        pltpu.make_async_copy(
            stage.at[p], out.at[0, :, pl.ds(0, BLK)], ssem.at[p]).wait()

    EU = 8

    def transpose(p):
        cb = [colb[p, pl.ds(g * L, L)] for g in range(BLK // L)]

        def ebody(eo, carry):
            for q in range(EU):
                e = eo * EU + q
                vals = [
                    plsc.load_gather(rows.at[p], [rowvecs[g], cb[g] + e])
                    for g in range(BLK // L)
                ]
                for g in range(BLK // L):
                    stage[p, e, pl.ds(g * L, L)] = vals[g]
            return carry

        lax.fori_loop(0, EMBED // EU, ebody, 0)

    for p in range(2):
        prep(p, p)

    def body(kk, carry):
        for p in range(2):
            k = 2 * kk + p
            pl.when(kk > 0)(lambda p=p: wait_store(p))
            wait_gather(p)
            transpose(p)
            fire_store(k, p)
            pl.when(kk < NBLK // 2 - 1)(lambda k=k, p=p: prep(k + 2, p))
        return carry

    lax.fori_loop(0, NBLK // 2, body, 0)
    for p in range(2):
        wait_store(p)


def kernel(src_indices, tgt_indices, src_table, tgt_table):
    src_idx = src_indices.T.astype(jnp.int32)
    tgt_idx = tgt_indices.T.astype(jnp.int32)
    src_tab = src_table.reshape(VOCAB // 2, 2 * EMBED)
    tgt_tab = tgt_table.reshape(VOCAB // 2, 2 * EMBED)
    src_out = _embed_lookup(src_tab, src_idx)
    tgt_out = _embed_lookup(tgt_tab, tgt_idx)
    return (
        jnp.transpose(src_out, (2, 0, 1)),
        jnp.transpose(tgt_out, (2, 0, 1)),
    )


# speedup vs baseline: 1.0314x; 1.0314x over previous
import functools

import jax
import jax.numpy as jnp
from jax import lax
from jax.experimental import pallas as pl
from jax.experimental.pallas import tpu as pltpu
from jax.experimental.pallas import tpu_sc as plsc

VOCAB = 1000000
EMBED = 64
BATCH = 16384
SEQ = 50

NC = 2
NS = 16
NW = NC * NS

BPW = BATCH // NW
BLK = 128
BLKS_PER_S = BPW // BLK
NBLK = SEQ * BLKS_PER_S
L = 16


@functools.partial(
    pl.kernel,
    out_type=jax.ShapeDtypeStruct((SEQ, EMBED, BATCH), jnp.float32),
    mesh=plsc.VectorSubcoreMesh(core_axis_name="c", subcore_axis_name="s"),
    scratch_types=[
        pltpu.VMEM((SEQ, BPW), jnp.int32),
        pltpu.VMEM((2, BLK, 128), jnp.float32),
        pltpu.VMEM((2, EMBED, BLK), jnp.float32),
        pltpu.VMEM((2, BLK), jnp.int32),
        pltpu.VMEM((2, BLK), jnp.int32),
        pltpu.SemaphoreType.DMA((2,)),
        pltpu.SemaphoreType.DMA((2,)),
    ],
    compiler_params=pltpu.CompilerParams(needs_layout_passes=False),
)
def _embed_lookup(tab2, idx_t, out, idxall, rows, stage, idxhalf, colb,
                  gsem, ssem):
    wid = lax.axis_index("s") * NC + lax.axis_index("c")
    w0 = wid * BPW
    iota16 = lax.iota(jnp.int32, L)
    rowvecs = [iota16 + g * L for g in range(BLK // L)]

    def stage_idx(s, carry):
        pltpu.sync_copy(idx_t.at[s, pl.ds(w0, BPW)], idxall.at[s])
        return carry

    lax.fori_loop(0, SEQ, stage_idx, 0)

    def prep(k, p):
        s = k // BLKS_PER_S
        c0 = (k % BLKS_PER_S) * BLK
        for g in range(BLK // L):
            v = idxall[s, pl.ds(c0 + g * L, L)]
            idxhalf[p, pl.ds(g * L, L)] = v >> 1
            colb[p, pl.ds(g * L, L)] = (v & 1) * EMBED
        pltpu.async_copy(tab2.at[idxhalf.at[p]], rows.at[p], gsem.at[p])

    def wait_gather(p):
        pltpu.make_async_copy(
            tab2.at[idxhalf.at[p]], rows.at[p], gsem.at[p]).wait()

    def fire_store(k, p):
        s = k // BLKS_PER_S
        b0 = w0 + (k % BLKS_PER_S) * BLK
        pltpu.async_copy(
            stage.at[p], out.at[s, :, pl.ds(b0, BLK)], ssem.at[p])

    def wait_store(p):
        pltpu.make_async_copy(
            stage.at[p], out.at[0, :, pl.ds(0, BLK)], ssem.at[p]).wait()

    EU = 8

    def transpose(p):
        cb = [colb[p, pl.ds(g * L, L)] for g in range(BLK // L)]

        def ebody(eo, carry):
            for q in range(0, EU, 2):
                e = eo * EU + q
                vals = [
                    plsc.load_gather(rows.at[p], [rowvecs[g], cb[g] + e + d])
                    for d in range(2)
                    for g in range(BLK // L)
                ]
                for d in range(2):
                    for g in range(BLK // L):
                        stage[p, e + d, pl.ds(g * L, L)] = (
                            vals[d * (BLK // L) + g])
            return carry

        lax.fori_loop(0, EMBED // EU, ebody, 0)

    for p in range(2):
        prep(p, p)

    def body(kk, carry):
        for p in range(2):
            k = 2 * kk + p
            pl.when(kk > 0)(lambda p=p: wait_store(p))
            wait_gather(p)
            transpose(p)
            fire_store(k, p)
            pl.when(kk < NBLK // 2 - 1)(lambda k=k, p=p: prep(k + 2, p))
        return carry

    lax.fori_loop(0, NBLK // 2, body, 0)
    for p in range(2):
        wait_store(p)


def kernel(src_indices, tgt_indices, src_table, tgt_table):
    src_idx = src_indices.T.astype(jnp.int32)
    tgt_idx = tgt_indices.T.astype(jnp.int32)
    src_tab = src_table.reshape(VOCAB // 2, 2 * EMBED)
    tgt_tab = tgt_table.reshape(VOCAB // 2, 2 * EMBED)
    src_out = _embed_lookup(src_tab, src_idx)
    tgt_out = _embed_lookup(tgt_tab, tgt_idx)
    return (
        jnp.transpose(src_out, (2, 0, 1)),
        jnp.transpose(tgt_out, (2, 0, 1)),
    )


# speedup vs baseline: 1.2621x; 1.2237x over previous
import functools

import jax
import jax.numpy as jnp
from jax import lax
from jax.experimental import pallas as pl
from jax.experimental.pallas import tpu as pltpu
from jax.experimental.pallas import tpu_sc as plsc

VOCAB = 1000000
EMBED = 64
BATCH = 16384
SEQ = 50

NC = 2
NS = 16
NW = NC * NS

TOTAL = BATCH * SEQ
PER_W = TOTAL // NW
CHUNK = 128
NCHUNK = PER_W // CHUNK

NB = 8
DEPTH = 4
GROUPS = (NCHUNK - 2 * DEPTH) // NB


@functools.partial(
    pl.kernel,
    out_type=jax.ShapeDtypeStruct((TOTAL, EMBED), jnp.float32),
    mesh=plsc.VectorSubcoreMesh(core_axis_name="c", subcore_axis_name="s"),
    scratch_types=[
        pltpu.VMEM((NCHUNK, CHUNK), jnp.int32),
        pltpu.VMEM((NB, CHUNK, EMBED), jnp.float32),
        pltpu.SemaphoreType.DMA((NB,)),
        pltpu.SemaphoreType.DMA((NB,)),
    ],
    compiler_params=pltpu.CompilerParams(use_tc_tiling_on_sc=False),
)
def _embed_lookup(table, idx_hbm, out, idx_v, rows, gsem, ssem):
    wid = lax.axis_index("s") * NC + lax.axis_index("c")
    row_base = wid * PER_W
    chunk_base = wid * NCHUNK

    pltpu.sync_copy(idx_hbm.at[pl.ds(chunk_base, NCHUNK)], idx_v)

    def fire_gather(b, j):
        pltpu.async_copy(table.at[idx_v.at[j]], rows.at[b], gsem.at[b])

    def wait_gather(b):
        pltpu.make_async_copy(
            table.at[idx_v.at[0]], rows.at[b], gsem.at[b]).wait()

    def fire_store(b, j):
        pltpu.async_copy(
            rows.at[b], out.at[pl.ds(row_base + j * CHUNK, CHUNK)],
            ssem.at[b])

    def wait_store(b):
        pltpu.make_async_copy(
            rows.at[b], out.at[pl.ds(row_base, CHUNK)], ssem.at[b]).wait()

    for b in range(DEPTH):
        fire_gather(b, b)
    for t in range(DEPTH):
        wait_gather(t)
        fire_store(t, t)
        fire_gather(t + DEPTH, t + DEPTH)

    def body(g, carry):
        j0 = DEPTH + g * NB
        for t in range(NB):
            j = j0 + t
            b_new = t
            b_cur = (t + DEPTH) % NB
            wait_store(b_new)
            fire_gather(b_new, j + DEPTH)
            wait_gather(b_cur)
            fire_store(b_cur, j)
        return carry

    lax.fori_loop(0, GROUPS, body, 0)

    for t in range(DEPTH):
        j = NCHUNK - DEPTH + t
        b = j % NB
        wait_gather(b)
        fire_store(b, j)
    for b in range(NB):
        wait_store(b)


def kernel(src_indices, tgt_indices, src_table, tgt_table):
    src_idx = src_indices.reshape(-1, CHUNK).astype(jnp.int32)
    tgt_idx = tgt_indices.reshape(-1, CHUNK).astype(jnp.int32)
    src_out = _embed_lookup(src_table, src_idx)
    tgt_out = _embed_lookup(tgt_table, tgt_idx)
    return (
        src_out.reshape(BATCH, SEQ, EMBED),
        tgt_out.reshape(BATCH, SEQ, EMBED),
    )
